# baseline (speedup 1.0000x reference)
import functools

import jax
import jax.numpy as jnp
from jax import lax
from jax.experimental import pallas as pl
from jax.experimental.pallas import tpu as pltpu
from jax.experimental.pallas import tpu_sc as plsc

N = 10000
NPAD = 10016
E = 320000
D = 128
DH = D // 2
NC = 2
NS = 16
CH = 128
EPT = E // NS
CPW = -(-EPT // CH)
PADE = CPW * CH - EPT
NB = 4
D0 = 79
ZR = NPAD // NS
RMAIN = 624
RTAIL = N - NS * RMAIN


def _sc_agg_body(with_deg, *refs):
    i = 5 if with_deg else 4
    if with_deg:
        feats, src3, dst3, parts, degp = refs[:5]
    else:
        feats, src3, dst3, parts = refs[:4]
    src_v, dst_v = refs[i:i + 2]
    i += 2
    rows = refs[i:i + NB]
    i += NB
    if with_deg:
        ones_v, z16_v = refs[i:i + 2]
        i += 2
    acc_sh = refs[i]
    i += 1
    deg_sh_ = None
    if with_deg:
        deg_sh_ = refs[i]
        i += 1
    gsem = refs[i:i + NB]
    i += NB
    ssem = refs[i:i + NB]
    i += NB
    dsem = refs[i] if with_deg else None

    c = lax.axis_index("c")
    s = lax.axis_index("s")

    fview = feats.at[pl.ds(c, 2 * N - 1)]

    pltpu.sync_copy(src3.at[pl.ds(s * CPW, CPW)], src_v)
    pltpu.sync_copy(dst3.at[pl.ds(s * CPW, CPW)], dst_v)

    zvec = jnp.zeros((16,), jnp.float32)

    @pl.loop(0, CH)
    def _(i):
        for k in range(DH // 16):
            rows[0][i, pl.ds(k * 16, 16)] = zvec

    if with_deg:
        ovec = jnp.ones((16,), jnp.float32)

        @pl.loop(0, CH)
        def _(i):
            ones_v[i, pl.ds(0, 16)] = ovec
            z16_v[i, pl.ds(0, 16)] = zvec

    for q in range(ZR // CH):
        pltpu.sync_copy(rows[0], acc_sh.at[pl.ds(s * ZR + q * CH, CH)])
        if with_deg:
            pltpu.sync_copy(z16_v, deg_sh_.at[pl.ds(s * ZR + q * CH, CH)])
    _REM = ZR % CH
    if _REM:
        _qb = s * ZR + (ZR // CH) * CH
        pltpu.sync_copy(rows[0].at[pl.ds(0, _REM)],
                        acc_sh.at[pl.ds(_qb, _REM)])
        if with_deg:
            pltpu.sync_copy(z16_v.at[pl.ds(0, _REM)],
                            deg_sh_.at[pl.ds(_qb, _REM)])

    plsc.subcore_barrier()

    def scat_issue(j, k):
        pltpu.async_copy(rows[k], acc_sh.at[dst_v.at[j]], ssem[k], add=True)
        if with_deg:
            do = (c == 0) == (j < D0)

            @pl.when(do)
            def _():
                pltpu.async_copy(ones_v, deg_sh_.at[dst_v.at[j]], dsem,
                                 add=True)

    def scat_wait(j, k):
        pltpu.make_async_copy(rows[k], acc_sh.at[dst_v.at[j]],
                              ssem[k]).wait()

    def gather(j, k):
        pltpu.async_copy(fview.at[src_v.at[j]], rows[k], gsem[k])

    def gwait(j, k):
        pltpu.make_async_copy(fview.at[src_v.at[j]], rows[k],
                              gsem[k]).wait()

    for k in range(NB):
        gather(k, k)

    @pl.loop(0, CPW, step=NB)
    def _(i):
        for k in range(NB):
            @pl.when(i + k < CPW)
            def _(k=k):
                gwait(i + k, k)
                scat_issue(i + k, k)
        for k in range(NB):
            @pl.when(i + k + NB < CPW)
            def _(k=k):
                scat_wait(i + k, k)
                gather(i + k + NB, k)

    for k in range(NB):
        scat_wait(0, k)

    if with_deg:
        ndeg = jnp.where(c == 0, D0, CPW - D0)

        @pl.loop(0, ndeg)
        def _(i):
            pltpu.make_async_copy(ones_v, deg_sh_.at[dst_v.at[0]],
                                  dsem).wait()

    plsc.subcore_barrier()

    base = s * RMAIN
    pltpu.sync_copy(acc_sh.at[pl.ds(base, RMAIN)],
                    parts.at[pl.ds(base, RMAIN), pl.ds(c * DH, DH)])
    if with_deg:
        pltpu.sync_copy(deg_sh_.at[pl.ds(base, RMAIN)],
                        degp.at[c, pl.ds(base, RMAIN)])

    @pl.when(s == NS - 1)
    def _():
        pltpu.sync_copy(acc_sh.at[pl.ds(NS * RMAIN, RTAIL)],
                        parts.at[pl.ds(NS * RMAIN, RTAIL),
                                 pl.ds(c * DH, DH)])
        if with_deg:
            pltpu.sync_copy(deg_sh_.at[pl.ds(NS * RMAIN, RTAIL)],
                            degp.at[c, pl.ds(NS * RMAIN, RTAIL)])


def _make_sc_agg(with_deg):
    mesh = plsc.VectorSubcoreMesh(core_axis_name="c", subcore_axis_name="s")
    out_type = [jax.ShapeDtypeStruct((N, D), jnp.float32)]
    if with_deg:
        out_type.append(jax.ShapeDtypeStruct((NC, N, 16), jnp.float32))
    scratch = [
        pltpu.VMEM((CPW, CH), jnp.int32),
        pltpu.VMEM((CPW, CH), jnp.int32),
    ]
    scratch += [pltpu.VMEM((CH, DH), jnp.float32) for _ in range(NB)]
    if with_deg:
        scratch += [pltpu.VMEM((CH, 16), jnp.float32),
                    pltpu.VMEM((CH, 16), jnp.float32)]
    scratch.append(pltpu.VMEM_SHARED((NPAD, DH), jnp.float32))
    if with_deg:
        scratch.append(pltpu.VMEM_SHARED((NPAD, 16), jnp.float32))
    scratch += [pltpu.SemaphoreType.DMA] * (2 * NB)
    if with_deg:
        scratch.append(pltpu.SemaphoreType.DMA)

    return pl.kernel(
        functools.partial(_sc_agg_body, with_deg),
        out_type=tuple(out_type) if len(out_type) > 1 else out_type[0],
        mesh=mesh,
        scratch_types=tuple(scratch),
        compiler_params=pltpu.CompilerParams(use_tc_tiling_on_sc=False),
    )


def _tc_mm_r_body(feat_ref, wr_ref, b_ref, out_ref):
    out_ref[...] = (lax.dot_general(feat_ref[...], wr_ref[...],
                                    (((1,), (1,)), ((), ())),
                                    preferred_element_type=jnp.float32)
                    + b_ref[...])


_tc_mm_r = pl.pallas_call(
    _tc_mm_r_body,
    out_shape=jax.ShapeDtypeStruct((N, D), jnp.float32),
)


def _tc_combine_body(act, p_ref, degp_ref, xr_ref, wl_ref, out_ref):
    deg = degp_ref[0, :, 0:1] + degp_ref[1, :, 0:1]
    agg = p_ref[...] / jnp.maximum(deg, 1.0)
    z = (lax.dot_general(agg, wl_ref[...], (((1,), (1,)), ((), ())),
                         preferred_element_type=jnp.float32)
         + xr_ref[...])
    if act == "relu":
        z = jnp.maximum(z, 0.0)
    else:
        m = jnp.max(z, axis=1, keepdims=True)
        z = z - (jnp.log(jnp.sum(jnp.exp(z - m), axis=1, keepdims=True)) + m)
    out_ref[...] = z


def _make_tc_combine(act):
    return pl.pallas_call(
        functools.partial(_tc_combine_body, act),
        out_shape=jax.ShapeDtypeStruct((N, D), jnp.float32),
    )


_sc_agg_deg = _make_sc_agg(True)
_sc_agg = _make_sc_agg(False)
_tc_relu = _make_tc_combine("relu")
_tc_lsm = _make_tc_combine("lsm")


def kernel(x, edge_index, W1_l, b1_l, W1_r, W2_l, b2_l, W2_r):
    ei = edge_index.astype(jnp.int32)
    src3 = jnp.pad((ei[0] * 2).reshape(NS, EPT),
                   ((0, 0), (0, PADE))).reshape(NS * CPW, CH)
    dst3 = jnp.pad(ei[1].reshape(NS, EPT), ((0, 0), (0, PADE)),
                   constant_values=N).reshape(NS * CPW, CH)
    b1 = b1_l.reshape(1, D)
    b2 = b2_l.reshape(1, D)

    xr1 = _tc_mm_r(x, W1_r, b1)
    p1, degp = _sc_agg_deg(x.reshape(NC * N, DH), src3, dst3)
    h = _tc_relu(p1, degp, xr1, W1_l)
    xr2 = _tc_mm_r(h, W2_r, b2)
    p2 = _sc_agg(h.reshape(NC * N, DH), src3, dst3)
    out = _tc_lsm(p2, degp, xr2, W2_l)
    return out


# speedup vs baseline: 1.0011x; 1.0011x over previous
import functools

import jax
import jax.numpy as jnp
from jax import lax
from jax.experimental import pallas as pl
from jax.experimental.pallas import tpu as pltpu
from jax.experimental.pallas import tpu_sc as plsc

N = 10000
NPAD = 10016
E = 320000
D = 128
DH = D // 2
NC = 2
NS = 16
CH = 128
EPT = E // NS
CPW = -(-EPT // CH)
PADE = CPW * CH - EPT
NB = 4
D0 = 79
ZR = NPAD // NS
RMAIN = 624
RTAIL = N - NS * RMAIN


def _sc_agg_body(with_deg, *refs):
    i = 5 if with_deg else 4
    if with_deg:
        feats, src3, dst3, parts, degp = refs[:5]
    else:
        feats, src3, dst3, parts = refs[:4]
    src_v, dst_v = refs[i:i + 2]
    i += 2
    rows = refs[i:i + NB]
    i += NB
    if with_deg:
        ones_v, z16_v = refs[i:i + 2]
        i += 2
    acc_sh = refs[i]
    i += 1
    deg_sh_ = None
    if with_deg:
        deg_sh_ = refs[i]
        i += 1
    gsem = refs[i:i + NB]
    i += NB
    ssem = refs[i:i + NB]
    i += NB
    dsem = refs[i] if with_deg else None

    c = lax.axis_index("c")
    s = lax.axis_index("s")

    fview = feats.at[pl.ds(c, 2 * N - 1)]

    pltpu.sync_copy(src3.at[pl.ds(s * CPW, CPW)], src_v)
    pltpu.sync_copy(dst3.at[pl.ds(s * CPW, CPW)], dst_v)

    zvec = jnp.zeros((16,), jnp.float32)

    @pl.loop(0, CH)
    def _(i):
        for k in range(DH // 16):
            rows[0][i, pl.ds(k * 16, 16)] = zvec

    if with_deg:
        ovec = jnp.ones((16,), jnp.float32)

        @pl.loop(0, CH)
        def _(i):
            ones_v[i, pl.ds(0, 16)] = ovec
            z16_v[i, pl.ds(0, 16)] = zvec

    for q in range(ZR // CH):
        pltpu.sync_copy(rows[0], acc_sh.at[pl.ds(s * ZR + q * CH, CH)])
        if with_deg:
            pltpu.sync_copy(z16_v, deg_sh_.at[pl.ds(s * ZR + q * CH, CH)])
    _REM = ZR % CH
    if _REM:
        _qb = s * ZR + (ZR // CH) * CH
        pltpu.sync_copy(rows[0].at[pl.ds(0, _REM)],
                        acc_sh.at[pl.ds(_qb, _REM)])
        if with_deg:
            pltpu.sync_copy(z16_v.at[pl.ds(0, _REM)],
                            deg_sh_.at[pl.ds(_qb, _REM)])

    plsc.subcore_barrier()

    def scat_issue(j, k):
        pltpu.async_copy(rows[k], acc_sh.at[dst_v.at[j]], ssem[k], add=True)
        if with_deg:
            do = (c == 0) == (j < D0)

            @pl.when(do)
            def _():
                pltpu.async_copy(ones_v, deg_sh_.at[dst_v.at[j]], dsem,
                                 add=True)

    def scat_wait(j, k):
        pltpu.make_async_copy(rows[k], acc_sh.at[dst_v.at[j]],
                              ssem[k]).wait()

    def gather(j, k):
        pltpu.async_copy(fview.at[src_v.at[j]], rows[k], gsem[k])

    def gwait(j, k):
        pltpu.make_async_copy(fview.at[src_v.at[j]], rows[k],
                              gsem[k]).wait()

    for k in range(NB):
        gather(k, k)

    @pl.loop(0, CPW, step=NB)
    def _(i):
        for k in range(NB):
            @pl.when(i + k < CPW)
            def _(k=k):
                gwait(i + k, k)
                scat_issue(i + k, k)
        for k in range(NB):
            @pl.when(i + k + NB < CPW)
            def _(k=k):
                scat_wait(i + k, k)
                gather(i + k + NB, k)

    for k in range(NB):
        scat_wait(0, k)

    if with_deg:
        ndeg = jnp.where(c == 0, D0, CPW - D0)

        @pl.loop(0, ndeg)
        def _(i):
            pltpu.make_async_copy(ones_v, deg_sh_.at[dst_v.at[0]],
                                  dsem).wait()

    plsc.subcore_barrier()

    base = s * RMAIN
    pltpu.sync_copy(acc_sh.at[pl.ds(base, RMAIN)],
                    parts.at[pl.ds(base, RMAIN), pl.ds(c * DH, DH)])
    if with_deg:
        pltpu.sync_copy(deg_sh_.at[pl.ds(base, RMAIN)],
                        degp.at[c, pl.ds(base, RMAIN)])

    @pl.when(s == NS - 1)
    def _():
        pltpu.sync_copy(acc_sh.at[pl.ds(NS * RMAIN, RTAIL)],
                        parts.at[pl.ds(NS * RMAIN, RTAIL),
                                 pl.ds(c * DH, DH)])
        if with_deg:
            pltpu.sync_copy(deg_sh_.at[pl.ds(NS * RMAIN, RTAIL)],
                            degp.at[c, pl.ds(NS * RMAIN, RTAIL)])


def _make_sc_agg(with_deg):
    mesh = plsc.VectorSubcoreMesh(core_axis_name="c", subcore_axis_name="s")
    out_type = [jax.ShapeDtypeStruct((N, D), jnp.float32)]
    if with_deg:
        out_type.append(jax.ShapeDtypeStruct((NC, N, 16), jnp.float32))
    scratch = [
        pltpu.VMEM((CPW, CH), jnp.int32),
        pltpu.VMEM((CPW, CH), jnp.int32),
    ]
    scratch += [pltpu.VMEM((CH, DH), jnp.float32) for _ in range(NB)]
    if with_deg:
        scratch += [pltpu.VMEM((CH, 16), jnp.float32),
                    pltpu.VMEM((CH, 16), jnp.float32)]
    scratch.append(pltpu.VMEM_SHARED((NPAD, DH), jnp.float32))
    if with_deg:
        scratch.append(pltpu.VMEM_SHARED((NPAD, 16), jnp.float32))
    scratch += [pltpu.SemaphoreType.DMA] * (2 * NB)
    if with_deg:
        scratch.append(pltpu.SemaphoreType.DMA)

    return pl.kernel(
        functools.partial(_sc_agg_body, with_deg),
        out_type=tuple(out_type) if len(out_type) > 1 else out_type[0],
        mesh=mesh,
        scratch_types=tuple(scratch),
        compiler_params=pltpu.CompilerParams(use_tc_tiling_on_sc=False),
    )


def _tc_mm_r_body(feat_ref, wr_ref, b_ref, out_ref):
    out_ref[...] = (lax.dot_general(feat_ref[...], wr_ref[...],
                                    (((1,), (1,)), ((), ())),
                                    preferred_element_type=jnp.float32)
                    + b_ref[...])


_tc_mm_r = pl.pallas_call(
    _tc_mm_r_body,
    out_shape=jax.ShapeDtypeStruct((N, D), jnp.float32),
)


def _tc_combine_body(act, p_ref, degp_ref, xr_ref, wl_ref, out_ref):
    deg = degp_ref[0, :, 0:1] + degp_ref[1, :, 0:1]
    agg = p_ref[...] / jnp.maximum(deg, 1.0)
    z = (lax.dot_general(agg, wl_ref[...], (((1,), (1,)), ((), ())),
                         preferred_element_type=jnp.float32)
         + xr_ref[...])
    if act == "relu":
        z = jnp.maximum(z, 0.0)
    else:
        m = jnp.max(z, axis=1, keepdims=True)
        z = z - (jnp.log(jnp.sum(jnp.exp(z - m), axis=1, keepdims=True)) + m)
    out_ref[...] = z


def _make_tc_combine(act):
    return pl.pallas_call(
        functools.partial(_tc_combine_body, act),
        out_shape=jax.ShapeDtypeStruct((N, D), jnp.float32),
    )


_sc_agg_deg = _make_sc_agg(True)
_sc_agg = _make_sc_agg(False)
_tc_relu = _make_tc_combine("relu")
_tc_lsm = _make_tc_combine("lsm")


def kernel(x, edge_index, W1_l, b1_l, W1_r, W2_l, b2_l, W2_r):
    ei = edge_index.astype(jnp.int32)
    src3 = jnp.pad((ei[0] * 2).reshape(NS, EPT),
                   ((0, 0), (0, PADE))).reshape(NS * CPW, CH)
    trash = jnp.broadcast_to(N + jnp.arange(NS, dtype=jnp.int32)[:, None],
                             (NS, PADE))
    dst3 = jnp.concatenate([ei[1].reshape(NS, EPT), trash],
                           axis=1).reshape(NS * CPW, CH)
    b1 = b1_l.reshape(1, D)
    b2 = b2_l.reshape(1, D)

    xr1 = _tc_mm_r(x, W1_r, b1)
    p1, degp = _sc_agg_deg(x.reshape(NC * N, DH), src3, dst3)
    h = _tc_relu(p1, degp, xr1, W1_l)
    xr2 = _tc_mm_r(h, W2_r, b2)
    p2 = _sc_agg(h.reshape(NC * N, DH), src3, dst3)
    out = _tc_lsm(p2, degp, xr2, W2_l)
    return out


# speedup vs baseline: 1.4108x; 1.4092x over previous
import functools

import jax
import jax.numpy as jnp
from jax import lax
from jax.experimental import pallas as pl
from jax.experimental.pallas import tpu as pltpu
from jax.experimental.pallas import tpu_sc as plsc

N = 10000
NPAD = N
E = 320000
D = 128
DH = D // 2
NC = 2
NS = 16
CH = 80
EPT = E // NS
CPW = EPT // CH
NB = 6
D0 = CPW // 2
ZR = NPAD // NS
RMAIN = 624
RTAIL = N - NS * RMAIN


def _sc_agg_body(with_deg, *refs):
    i = 5 if with_deg else 4
    if with_deg:
        feats, src3, dst3, parts, degp = refs[:5]
    else:
        feats, src3, dst3, parts = refs[:4]
    src_v, dst_v = refs[i:i + 2]
    i += 2
    rows = refs[i:i + NB]
    i += NB
    if with_deg:
        ones_v, z16_v = refs[i:i + 2]
        i += 2
    acc_sh = refs[i]
    i += 1
    deg_sh_ = None
    if with_deg:
        deg_sh_ = refs[i]
        i += 1
    gsem = refs[i:i + NB]
    i += NB
    ssem = refs[i:i + NB]
    i += NB
    dsem = refs[i] if with_deg else None

    c = lax.axis_index("c")
    s = lax.axis_index("s")

    fview = feats.at[pl.ds(c, 2 * N - 1)]

    pltpu.sync_copy(src3.at[s], src_v)
    pltpu.sync_copy(dst3.at[s], dst_v)

    zvec = jnp.zeros((16,), jnp.float32)

    @pl.loop(0, CH)
    def _(i):
        for k in range(DH // 16):
            rows[0][i, pl.ds(k * 16, 16)] = zvec

    if with_deg:
        ovec = jnp.ones((16,), jnp.float32)

        @pl.loop(0, CH)
        def _(i):
            ones_v[i, pl.ds(0, 16)] = ovec
            z16_v[i, pl.ds(0, 16)] = zvec

    for q in range(ZR // CH):
        pltpu.sync_copy(rows[0], acc_sh.at[pl.ds(s * ZR + q * CH, CH)])
        if with_deg:
            pltpu.sync_copy(z16_v, deg_sh_.at[pl.ds(s * ZR + q * CH, CH)])
    _REM = ZR % CH
    if _REM:
        _qb = s * ZR + (ZR // CH) * CH
        pltpu.sync_copy(rows[0].at[pl.ds(0, _REM)],
                        acc_sh.at[pl.ds(_qb, _REM)])
        if with_deg:
            pltpu.sync_copy(z16_v.at[pl.ds(0, _REM)],
                            deg_sh_.at[pl.ds(_qb, _REM)])

    plsc.subcore_barrier()

    def scat_issue(j, k):
        pltpu.async_copy(rows[k], acc_sh.at[dst_v.at[j]], ssem[k], add=True)
        if with_deg:
            do = (c == 0) == (j < D0)

            @pl.when(do)
            def _():
                pltpu.async_copy(ones_v, deg_sh_.at[dst_v.at[j]], dsem,
                                 add=True)

    def scat_wait(j, k):
        pltpu.make_async_copy(rows[k], acc_sh.at[dst_v.at[j]],
                              ssem[k]).wait()

    def gather(j, k):
        pltpu.async_copy(fview.at[src_v.at[j]], rows[k], gsem[k])

    def gwait(j, k):
        pltpu.make_async_copy(fview.at[src_v.at[j]], rows[k],
                              gsem[k]).wait()

    for k in range(NB):
        gather(k, k)

    @pl.loop(0, CPW, step=NB)
    def _(i):
        for k in range(NB):
            @pl.when(i + k < CPW)
            def _(k=k):
                gwait(i + k, k)
                scat_issue(i + k, k)
        for k in range(NB):
            @pl.when(i + k + NB < CPW)
            def _(k=k):
                scat_wait(i + k, k)
                gather(i + k + NB, k)

    for k in range(NB):
        scat_wait(0, k)

    if with_deg:
        @pl.loop(0, D0)
        def _(i):
            pltpu.make_async_copy(ones_v, deg_sh_.at[dst_v.at[0]],
                                  dsem).wait()

    plsc.subcore_barrier()

    base = s * RMAIN
    pltpu.sync_copy(acc_sh.at[pl.ds(base, RMAIN)],
                    parts.at[pl.ds(base, RMAIN), pl.ds(c * DH, DH)])
    if with_deg:
        pltpu.sync_copy(deg_sh_.at[pl.ds(base, RMAIN)],
                        degp.at[c, pl.ds(base, RMAIN)])

    @pl.when(s == NS - 1)
    def _():
        pltpu.sync_copy(acc_sh.at[pl.ds(NS * RMAIN, RTAIL)],
                        parts.at[pl.ds(NS * RMAIN, RTAIL),
                                 pl.ds(c * DH, DH)])
        if with_deg:
            pltpu.sync_copy(deg_sh_.at[pl.ds(NS * RMAIN, RTAIL)],
                            degp.at[c, pl.ds(NS * RMAIN, RTAIL)])


def _make_sc_agg(with_deg):
    mesh = plsc.VectorSubcoreMesh(core_axis_name="c", subcore_axis_name="s")
    out_type = [jax.ShapeDtypeStruct((N, D), jnp.float32)]
    if with_deg:
        out_type.append(jax.ShapeDtypeStruct((NC, N, 16), jnp.float32))
    scratch = [
        pltpu.VMEM((CPW, CH), jnp.int32),
        pltpu.VMEM((CPW, CH), jnp.int32),
    ]
    scratch += [pltpu.VMEM((CH, DH), jnp.float32) for _ in range(NB)]
    if with_deg:
        scratch += [pltpu.VMEM((CH, 16), jnp.float32),
                    pltpu.VMEM((CH, 16), jnp.float32)]
    scratch.append(pltpu.VMEM_SHARED((NPAD, DH), jnp.float32))
    if with_deg:
        scratch.append(pltpu.VMEM_SHARED((NPAD, 16), jnp.float32))
    scratch += [pltpu.SemaphoreType.DMA] * (2 * NB)
    if with_deg:
        scratch.append(pltpu.SemaphoreType.DMA)

    return pl.kernel(
        functools.partial(_sc_agg_body, with_deg),
        out_type=tuple(out_type) if len(out_type) > 1 else out_type[0],
        mesh=mesh,
        scratch_types=tuple(scratch),
        compiler_params=pltpu.CompilerParams(use_tc_tiling_on_sc=False),
    )


def _tc_mm_r_body(feat_ref, wr_ref, b_ref, out_ref):
    out_ref[...] = (lax.dot_general(feat_ref[...], wr_ref[...],
                                    (((1,), (1,)), ((), ())),
                                    preferred_element_type=jnp.float32)
                    + b_ref[...])


_tc_mm_r = pl.pallas_call(
    _tc_mm_r_body,
    out_shape=jax.ShapeDtypeStruct((N, D), jnp.float32),
)


def _tc_combine_body(act, p_ref, degp_ref, xr_ref, wl_ref, out_ref):
    deg = degp_ref[0, :, 0:1] + degp_ref[1, :, 0:1]
    agg = p_ref[...] / jnp.maximum(deg, 1.0)
    z = (lax.dot_general(agg, wl_ref[...], (((1,), (1,)), ((), ())),
                         preferred_element_type=jnp.float32)
         + xr_ref[...])
    if act == "relu":
        z = jnp.maximum(z, 0.0)
    else:
        m = jnp.max(z, axis=1, keepdims=True)
        z = z - (jnp.log(jnp.sum(jnp.exp(z - m), axis=1, keepdims=True)) + m)
    out_ref[...] = z


def _make_tc_combine(act):
    return pl.pallas_call(
        functools.partial(_tc_combine_body, act),
        out_shape=jax.ShapeDtypeStruct((N, D), jnp.float32),
    )


_sc_agg_deg = _make_sc_agg(True)
_sc_agg = _make_sc_agg(False)
_tc_relu = _make_tc_combine("relu")
_tc_lsm = _make_tc_combine("lsm")


def kernel(x, edge_index, W1_l, b1_l, W1_r, W2_l, b2_l, W2_r):
    ei = edge_index.astype(jnp.int32)
    src3 = (ei[0] * 2).reshape(NS, CPW, CH)
    dst3 = ei[1].reshape(NS, CPW, CH)
    b1 = b1_l.reshape(1, D)
    b2 = b2_l.reshape(1, D)

    xr1 = _tc_mm_r(x, W1_r, b1)
    p1, degp = _sc_agg_deg(x.reshape(NC * N, DH), src3, dst3)
    h = _tc_relu(p1, degp, xr1, W1_l)
    xr2 = _tc_mm_r(h, W2_r, b2)
    p2 = _sc_agg(h.reshape(NC * N, DH), src3, dst3)
    out = _tc_lsm(p2, degp, xr2, W2_l)
    return out


# speedup vs baseline: 1.4491x; 1.0272x over previous
import functools

import jax
import jax.numpy as jnp
from jax import lax
from jax.experimental import pallas as pl
from jax.experimental.pallas import tpu as pltpu
from jax.experimental.pallas import tpu_sc as plsc

N = 10000
NPAD = N
E = 320000
D = 128
DH = D // 2
NC = 2
NS = 16
CH = 80
EPT = E // NS
CPW = EPT // CH
NB = 6
D0 = CPW // 2
ZR = NPAD // NS
RMAIN = 624
RTAIL = N - NS * RMAIN


def _sc_agg_body(with_deg, *refs):
    i = 5 if with_deg else 4
    if with_deg:
        feats, src3, dst3, parts, degp = refs[:5]
    else:
        feats, src3, dst3, parts = refs[:4]
    src_v, dst_v = refs[i:i + 2]
    i += 2
    rows = refs[i:i + NB]
    i += NB
    if with_deg:
        ones_v, z16_v = refs[i:i + 2]
        i += 2
    acc_sh = refs[i]
    i += 1
    deg_sh_ = None
    if with_deg:
        deg_sh_ = refs[i]
        i += 1
    gsem = refs[i:i + NB]
    i += NB
    ssem = refs[i:i + NB]
    i += NB
    dsem = refs[i] if with_deg else None

    c = lax.axis_index("c")
    s = lax.axis_index("s")

    fview = feats.at[pl.ds(c, 2 * N - 1)]

    pltpu.async_copy(src3.at[s], src_v, gsem[0])
    pltpu.async_copy(dst3.at[s], dst_v, gsem[1])

    zvec = jnp.zeros((16,), jnp.float32)

    @pl.loop(0, CH)
    def _(i):
        for k in range(DH // 16):
            rows[0][i, pl.ds(k * 16, 16)] = zvec

    if with_deg:
        ovec = jnp.ones((16,), jnp.float32)

        @pl.loop(0, CH)
        def _(i):
            ones_v[i, pl.ds(0, 16)] = ovec
            z16_v[i, pl.ds(0, 16)] = zvec

    for q in range(ZR // CH):
        pltpu.async_copy(rows[0], acc_sh.at[pl.ds(s * ZR + q * CH, CH)],
                         ssem[0])
        if with_deg:
            pltpu.async_copy(z16_v, deg_sh_.at[pl.ds(s * ZR + q * CH, CH)],
                             ssem[1])
    _REM = ZR % CH
    if _REM:
        _qb = s * ZR + (ZR // CH) * CH
        pltpu.async_copy(rows[0].at[pl.ds(0, _REM)],
                         acc_sh.at[pl.ds(_qb, _REM)], ssem[0])
        if with_deg:
            pltpu.async_copy(z16_v.at[pl.ds(0, _REM)],
                             deg_sh_.at[pl.ds(_qb, _REM)], ssem[1])

    pltpu.make_async_copy(src3.at[s], src_v, gsem[0]).wait()
    pltpu.make_async_copy(dst3.at[s], dst_v, gsem[1]).wait()
    for q in range(ZR // CH):
        pltpu.make_async_copy(rows[0], acc_sh.at[pl.ds(s * ZR, CH)],
                              ssem[0]).wait()
        if with_deg:
            pltpu.make_async_copy(z16_v, deg_sh_.at[pl.ds(s * ZR, CH)],
                                  ssem[1]).wait()
    if _REM:
        pltpu.make_async_copy(rows[0].at[pl.ds(0, _REM)],
                              acc_sh.at[pl.ds(s * ZR, _REM)], ssem[0]).wait()
        if with_deg:
            pltpu.make_async_copy(z16_v.at[pl.ds(0, _REM)],
                                  deg_sh_.at[pl.ds(s * ZR, _REM)],
                                  ssem[1]).wait()

    plsc.subcore_barrier()

    def scat_issue(j, k):
        pltpu.async_copy(rows[k], acc_sh.at[dst_v.at[j]], ssem[k], add=True)
        if with_deg:
            do = (c == 0) == (j < D0)

            @pl.when(do)
            def _():
                pltpu.async_copy(ones_v, deg_sh_.at[dst_v.at[j]], dsem,
                                 add=True)

    def scat_wait(j, k):
        pltpu.make_async_copy(rows[k], acc_sh.at[dst_v.at[j]],
                              ssem[k]).wait()

    def gather(j, k):
        pltpu.async_copy(fview.at[src_v.at[j]], rows[k], gsem[k])

    def gwait(j, k):
        pltpu.make_async_copy(fview.at[src_v.at[j]], rows[k],
                              gsem[k]).wait()

    for k in range(NB):
        gather(k, k)

    @pl.loop(0, CPW, step=NB)
    def _(i):
        for k in range(NB):
            @pl.when(i + k < CPW)
            def _(k=k):
                gwait(i + k, k)
                scat_issue(i + k, k)
        for k in range(NB):
            @pl.when(i + k + NB < CPW)
            def _(k=k):
                scat_wait(i + k, k)
                gather(i + k + NB, k)

    for k in range(NB):
        scat_wait(0, k)

    if with_deg:
        @pl.loop(0, D0)
        def _(i):
            pltpu.make_async_copy(ones_v, deg_sh_.at[dst_v.at[0]],
                                  dsem).wait()

    plsc.subcore_barrier()

    base = s * RMAIN
    pltpu.sync_copy(acc_sh.at[pl.ds(base, RMAIN)],
                    parts.at[pl.ds(base, RMAIN), pl.ds(c * DH, DH)])
    if with_deg:
        pltpu.sync_copy(deg_sh_.at[pl.ds(base, RMAIN)],
                        degp.at[c, pl.ds(base, RMAIN)])

    @pl.when(s == NS - 1)
    def _():
        pltpu.sync_copy(acc_sh.at[pl.ds(NS * RMAIN, RTAIL)],
                        parts.at[pl.ds(NS * RMAIN, RTAIL),
                                 pl.ds(c * DH, DH)])
        if with_deg:
            pltpu.sync_copy(deg_sh_.at[pl.ds(NS * RMAIN, RTAIL)],
                            degp.at[c, pl.ds(NS * RMAIN, RTAIL)])


def _make_sc_agg(with_deg):
    mesh = plsc.VectorSubcoreMesh(core_axis_name="c", subcore_axis_name="s")
    out_type = [jax.ShapeDtypeStruct((N, D), jnp.float32)]
    if with_deg:
        out_type.append(jax.ShapeDtypeStruct((NC, N, 16), jnp.float32))
    scratch = [
        pltpu.VMEM((CPW, CH), jnp.int32),
        pltpu.VMEM((CPW, CH), jnp.int32),
    ]
    scratch += [pltpu.VMEM((CH, DH), jnp.float32) for _ in range(NB)]
    if with_deg:
        scratch += [pltpu.VMEM((CH, 16), jnp.float32),
                    pltpu.VMEM((CH, 16), jnp.float32)]
    scratch.append(pltpu.VMEM_SHARED((NPAD, DH), jnp.float32))
    if with_deg:
        scratch.append(pltpu.VMEM_SHARED((NPAD, 16), jnp.float32))
    scratch += [pltpu.SemaphoreType.DMA] * (2 * NB)
    if with_deg:
        scratch.append(pltpu.SemaphoreType.DMA)

    return pl.kernel(
        functools.partial(_sc_agg_body, with_deg),
        out_type=tuple(out_type) if len(out_type) > 1 else out_type[0],
        mesh=mesh,
        scratch_types=tuple(scratch),
        compiler_params=pltpu.CompilerParams(use_tc_tiling_on_sc=False),
    )


def _tc_mm_r_body(feat_ref, wr_ref, b_ref, out_ref):
    out_ref[...] = (lax.dot_general(feat_ref[...], wr_ref[...],
                                    (((1,), (1,)), ((), ())),
                                    preferred_element_type=jnp.float32)
                    + b_ref[...])


_tc_mm_r = pl.pallas_call(
    _tc_mm_r_body,
    out_shape=jax.ShapeDtypeStruct((N, D), jnp.float32),
)


def _tc_combine_body(act, p_ref, degp_ref, xr_ref, wl_ref, out_ref):
    deg = degp_ref[0, :, 0:1] + degp_ref[1, :, 0:1]
    agg = p_ref[...] / jnp.maximum(deg, 1.0)
    z = (lax.dot_general(agg, wl_ref[...], (((1,), (1,)), ((), ())),
                         preferred_element_type=jnp.float32)
         + xr_ref[...])
    if act == "relu":
        z = jnp.maximum(z, 0.0)
    else:
        m = jnp.max(z, axis=1, keepdims=True)
        z = z - (jnp.log(jnp.sum(jnp.exp(z - m), axis=1, keepdims=True)) + m)
    out_ref[...] = z


def _make_tc_combine(act):
    return pl.pallas_call(
        functools.partial(_tc_combine_body, act),
        out_shape=jax.ShapeDtypeStruct((N, D), jnp.float32),
    )


_sc_agg_deg = _make_sc_agg(True)
_sc_agg = _make_sc_agg(False)
_tc_relu = _make_tc_combine("relu")
_tc_lsm = _make_tc_combine("lsm")


def kernel(x, edge_index, W1_l, b1_l, W1_r, W2_l, b2_l, W2_r):
    ei = edge_index.astype(jnp.int32)
    src3 = (ei[0] * 2).reshape(NS, CPW, CH)
    dst3 = ei[1].reshape(NS, CPW, CH)
    b1 = b1_l.reshape(1, D)
    b2 = b2_l.reshape(1, D)

    xr1 = _tc_mm_r(x, W1_r, b1)
    p1, degp = _sc_agg_deg(x.reshape(NC * N, DH), src3, dst3)
    h = _tc_relu(p1, degp, xr1, W1_l)
    xr2 = _tc_mm_r(h, W2_r, b2)
    p2 = _sc_agg(h.reshape(NC * N, DH), src3, dst3)
    out = _tc_lsm(p2, degp, xr2, W2_l)
    return out
